# Initial kernel scaffold; baseline (speedup 1.0000x reference)
import jax
import jax.numpy as jnp
from jax.experimental import pallas as pl


def kernel(z, prev_symbol_idx, codebook, adjacency):
    raise NotImplementedError("write your pallas kernel here")


# speedup vs baseline: 1.1665x; 1.1665x over previous
import functools

import jax
import jax.numpy as jnp
from jax import lax
from jax.experimental import pallas as pl
from jax.experimental.pallas import tpu as pltpu
from jax.experimental.pallas import tpu_sc as plsc

_BIAS_SCALE = 0.8
_COMMIT = 0.01

_B, _T, _LATENT, _K = 32, 1024, 128, 1024
_D = 2 * _LATENT
_TOK = _B * _T
_TOK_TILE = 256
_N_TILES = _TOK // _TOK_TILE

_NC, _NS = 2, 16
_NW = _NC * _NS
_BPW = _TOK // _NW
_CH = 64
_NCH = _BPW // _CH


def _prep_body(adj_ref, cb_ref, s_ref, cnorm_ref, hi_ref, mid_ref, lo_ref):
    adj = adj_ref[...]
    s_ref[...] = _BIAS_SCALE * jax.nn.sigmoid(adj)
    cb = cb_ref[...]
    q = cb * cb
    ones = jnp.ones((1, _D), jnp.float32)
    cnorm_ref[...] = lax.dot_general(
        ones, q, (((1,), (1,)), ((), ())),
        precision=lax.Precision.HIGHEST,
        preferred_element_type=jnp.float32)
    hi = cb.astype(jnp.bfloat16)
    r1 = cb - hi.astype(jnp.float32)
    mid = r1.astype(jnp.bfloat16)
    r2 = r1 - mid.astype(jnp.float32)
    hi_ref[...] = hi
    mid_ref[...] = mid
    lo_ref[...] = r2.astype(jnp.bfloat16)


def _prep(adjacency, codebook):
    return pl.pallas_call(
        _prep_body,
        out_shape=(
            jax.ShapeDtypeStruct((_K, _K), jnp.float32),
            jax.ShapeDtypeStruct((1, _K), jnp.float32),
            jax.ShapeDtypeStruct((_K, _D), jnp.bfloat16),
            jax.ShapeDtypeStruct((_K, _D), jnp.bfloat16),
            jax.ShapeDtypeStruct((_K, _D), jnp.bfloat16),
        ),
    )(adjacency, codebook)


@functools.partial(
    pl.kernel,
    mesh=plsc.VectorSubcoreMesh(core_axis_name="c", subcore_axis_name="s"),
    out_type=jax.ShapeDtypeStruct((_TOK, _K), jnp.float32),
    scratch_types=[
        pltpu.VMEM((_CH,), jnp.int32),
        pltpu.VMEM((_CH, _K), jnp.float32),
        pltpu.SemaphoreType.DMA,
    ],
)
def _sc_gather(s_hbm, idx_hbm, out_hbm, idx_v, rows_v, sem):
    wid = lax.axis_index("s") * _NC + lax.axis_index("c")
    base = wid * _BPW

    def body(c, carry):
        off = base + c * _CH
        pltpu.sync_copy(idx_hbm.at[pl.ds(off, _CH)], idx_v)
        pltpu.async_copy(s_hbm.at[idx_v], rows_v, sem).wait()
        pltpu.sync_copy(rows_v, out_hbm.at[pl.ds(off, _CH)])
        return carry

    lax.fori_loop(0, _NCH, body, 0)


def _main_body(z_ref, bias_ref, cb_ref, cnorm_ref, hi_ref, mid_ref, lo_ref,
               idx_ref, zq_ref, loss_ref):
    zt = z_ref[...]
    m2 = lax.dot_general(
        zt, cb_ref[...], (((1,), (1,)), ((), ())),
        preferred_element_type=jnp.float32)
    zsum = jnp.sum(zt * zt, axis=1, keepdims=True)
    d = (zsum + cnorm_ref[...]) - 2.0 * m2
    d = d - bias_ref[...]

    dmin = jnp.min(d, axis=1, keepdims=True)
    iota = lax.broadcasted_iota(jnp.int32, (_TOK_TILE, _K), 1)
    masked = jnp.where(d == dmin, iota, _K)
    idx = jnp.min(masked, axis=1, keepdims=True)
    idx_ref[...] = idx

    idxb = jnp.broadcast_to(idx, (_TOK_TILE, _K))
    onehot = jnp.where(iota == idxb, 1.0, 0.0).astype(jnp.bfloat16)
    dims = (((1,), (0,)), ((), ()))
    zq = (lax.dot_general(onehot, hi_ref[...], dims,
                          preferred_element_type=jnp.float32)
          + lax.dot_general(onehot, mid_ref[...], dims,
                            preferred_element_type=jnp.float32))
    zq = zq + lax.dot_general(onehot, lo_ref[...], dims,
                              preferred_element_type=jnp.float32)
    zq_ref[...] = zq

    diff = zq - zt
    part = jnp.sum(diff * diff, keepdims=True)

    @pl.when(pl.program_id(0) == 0)
    def _():
        loss_ref[...] = jnp.zeros((1, 1), jnp.float32)

    loss_ref[...] += part


def _main(z_flat, bias, codebook, cnorm, hi, mid, lo):
    return pl.pallas_call(
        _main_body,
        grid=(_N_TILES,),
        in_specs=[
            pl.BlockSpec((_TOK_TILE, _D), lambda i: (i, 0)),
            pl.BlockSpec((_TOK_TILE, _K), lambda i: (i, 0)),
            pl.BlockSpec((_K, _D), lambda i: (0, 0)),
            pl.BlockSpec((1, _K), lambda i: (0, 0)),
            pl.BlockSpec((_K, _D), lambda i: (0, 0)),
            pl.BlockSpec((_K, _D), lambda i: (0, 0)),
            pl.BlockSpec((_K, _D), lambda i: (0, 0)),
        ],
        out_specs=[
            pl.BlockSpec((_TOK_TILE, 1), lambda i: (i, 0)),
            pl.BlockSpec((_TOK_TILE, _D), lambda i: (i, 0)),
            pl.BlockSpec((1, 1), lambda i: (0, 0)),
        ],
        out_shape=(
            jax.ShapeDtypeStruct((_TOK, 1), jnp.int32),
            jax.ShapeDtypeStruct((_TOK, _D), jnp.float32),
            jax.ShapeDtypeStruct((1, 1), jnp.float32),
        ),
    )(z_flat, bias, codebook, cnorm, hi, mid, lo)


def kernel(z, prev_symbol_idx, codebook, adjacency):
    z_flat = jnp.concatenate([z.real, z.imag], axis=-1).reshape(_TOK, _D)
    prev = prev_symbol_idx.reshape(_TOK).astype(jnp.int32)

    s, cnorm, hi, mid, lo = _prep(adjacency, codebook)
    bias = _sc_gather(s, prev)
    idx, zq, loss_sum = _main(z_flat, bias, codebook, cnorm, hi, mid, lo)

    mean = loss_sum[0, 0] / jnp.float32(_TOK * _D)
    loss = mean + jnp.float32(_COMMIT) * mean

    zq3 = zq.reshape(_B, _T, _D)
    z_out = lax.complex(zq3[..., :_LATENT], zq3[..., _LATENT:])
    return (z_out, loss, idx.reshape(_B, _T))


# speedup vs baseline: 1.2952x; 1.1103x over previous
import functools

import jax
import jax.numpy as jnp
from jax import lax
from jax.experimental import pallas as pl
from jax.experimental.pallas import tpu as pltpu
from jax.experimental.pallas import tpu_sc as plsc

_BIAS_SCALE = 0.8
_COMMIT = 0.01

_B, _T, _LATENT, _K = 32, 1024, 128, 1024
_D = 2 * _LATENT
_TOK = _B * _T
_TOK_TILE = 512
_N_TILES = _TOK // _TOK_TILE

_NC, _NS = 2, 16
_NW = _NC * _NS
_BPW = _TOK // _NW
_CH = 64
_NCH = _BPW // _CH


def _prep_body(adj_ref, cb_ref, s_ref, cnorm_ref, hi_ref, mid_ref, lo_ref):
    adj = adj_ref[...]
    s_ref[...] = _BIAS_SCALE * jax.nn.sigmoid(adj)
    cb = cb_ref[...]
    q = cb * cb
    ones = jnp.ones((1, _D), jnp.float32)
    cnorm_ref[...] = lax.dot_general(
        ones, q, (((1,), (1,)), ((), ())),
        precision=lax.Precision.HIGHEST,
        preferred_element_type=jnp.float32)
    hi = cb.astype(jnp.bfloat16)
    r1 = cb - hi.astype(jnp.float32)
    mid = r1.astype(jnp.bfloat16)
    r2 = r1 - mid.astype(jnp.float32)
    hi_ref[...] = hi
    mid_ref[...] = mid
    lo_ref[...] = r2.astype(jnp.bfloat16)


def _prep(adjacency, codebook):
    return pl.pallas_call(
        _prep_body,
        out_shape=(
            jax.ShapeDtypeStruct((_K, _K), jnp.float32),
            jax.ShapeDtypeStruct((1, _K), jnp.float32),
            jax.ShapeDtypeStruct((_K, _D), jnp.bfloat16),
            jax.ShapeDtypeStruct((_K, _D), jnp.bfloat16),
            jax.ShapeDtypeStruct((_K, _D), jnp.bfloat16),
        ),
    )(adjacency, codebook)


@functools.partial(
    pl.kernel,
    mesh=plsc.VectorSubcoreMesh(core_axis_name="c", subcore_axis_name="s"),
    out_type=jax.ShapeDtypeStruct((_TOK, _K), jnp.float32),
    scratch_types=[
        pltpu.VMEM((_CH,), jnp.int32),
        pltpu.VMEM((_CH, _K), jnp.float32),
        pltpu.SemaphoreType.DMA,
    ],
)
def _sc_gather(s_hbm, idx_hbm, out_hbm, idx_v, rows_v, sem):
    wid = lax.axis_index("s") * _NC + lax.axis_index("c")
    base = wid * _BPW

    def body(c, carry):
        off = base + c * _CH
        pltpu.sync_copy(idx_hbm.at[pl.ds(off, _CH)], idx_v)
        pltpu.async_copy(s_hbm.at[idx_v], rows_v, sem).wait()
        pltpu.sync_copy(rows_v, out_hbm.at[pl.ds(off, _CH)])
        return carry

    lax.fori_loop(0, _NCH, body, 0)


def _main_body(zre_ref, zim_ref, bias_ref, cb_ref, cnorm_ref,
               hi_ref, mid_ref, lo_ref,
               idx_ref, zqre_ref, zqim_ref, loss_ref):
    zt = jnp.concatenate([zre_ref[...], zim_ref[...]], axis=1)
    m2 = lax.dot_general(
        zt, cb_ref[...], (((1,), (1,)), ((), ())),
        preferred_element_type=jnp.float32)
    zsum = jnp.sum(zt * zt, axis=1, keepdims=True)
    bt = bias_ref[...]
    d = (zsum + cnorm_ref[...]) - 2.0 * m2
    d = d - bt

    dmin = jnp.min(d, axis=1, keepdims=True)
    iota = lax.broadcasted_iota(jnp.int32, (_TOK_TILE, _K), 1)
    masked = jnp.where(d == dmin, iota, _K)
    idx = jnp.min(masked, axis=1, keepdims=True)
    idx_ref[...] = idx

    idxb = jnp.broadcast_to(idx, (_TOK_TILE, _K))
    onehot = jnp.where(iota == idxb, 1.0, 0.0)
    bias_at = jnp.sum(onehot * bt, axis=1, keepdims=True)
    lpart = jnp.sum(dmin + bias_at, keepdims=True)
    onehot_bf = onehot.astype(jnp.bfloat16)
    dims = (((1,), (0,)), ((), ()))
    zq = (lax.dot_general(onehot_bf, hi_ref[...], dims,
                          preferred_element_type=jnp.float32)
          + lax.dot_general(onehot_bf, mid_ref[...], dims,
                            preferred_element_type=jnp.float32))
    zq = zq + lax.dot_general(onehot_bf, lo_ref[...], dims,
                              preferred_element_type=jnp.float32)
    zqre_ref[...] = zq[:, :_LATENT]
    zqim_ref[...] = zq[:, _LATENT:]

    @pl.when(pl.program_id(0) == 0)
    def _():
        loss_ref[...] = jnp.zeros((1, 1), jnp.float32)

    loss_ref[...] += lpart


def _main(z_re, z_im, bias, codebook, cnorm, hi, mid, lo):
    return pl.pallas_call(
        _main_body,
        grid=(_N_TILES,),
        in_specs=[
            pl.BlockSpec((_TOK_TILE, _LATENT), lambda i: (i, 0)),
            pl.BlockSpec((_TOK_TILE, _LATENT), lambda i: (i, 0)),
            pl.BlockSpec((_TOK_TILE, _K), lambda i: (i, 0)),
            pl.BlockSpec((_K, _D), lambda i: (0, 0)),
            pl.BlockSpec((1, _K), lambda i: (0, 0)),
            pl.BlockSpec((_K, _D), lambda i: (0, 0)),
            pl.BlockSpec((_K, _D), lambda i: (0, 0)),
            pl.BlockSpec((_K, _D), lambda i: (0, 0)),
        ],
        out_specs=[
            pl.BlockSpec((_TOK_TILE, 1), lambda i: (i, 0)),
            pl.BlockSpec((_TOK_TILE, _LATENT), lambda i: (i, 0)),
            pl.BlockSpec((_TOK_TILE, _LATENT), lambda i: (i, 0)),
            pl.BlockSpec((1, 1), lambda i: (0, 0)),
        ],
        out_shape=(
            jax.ShapeDtypeStruct((_TOK, 1), jnp.int32),
            jax.ShapeDtypeStruct((_TOK, _LATENT), jnp.float32),
            jax.ShapeDtypeStruct((_TOK, _LATENT), jnp.float32),
            jax.ShapeDtypeStruct((1, 1), jnp.float32),
        ),
    )(z_re, z_im, bias, codebook, cnorm, hi, mid, lo)


def kernel(z, prev_symbol_idx, codebook, adjacency):
    z_re = z.real.reshape(_TOK, _LATENT)
    z_im = z.imag.reshape(_TOK, _LATENT)
    prev = prev_symbol_idx.reshape(_TOK).astype(jnp.int32)

    s, cnorm, hi, mid, lo = _prep(adjacency, codebook)
    bias = _sc_gather(s, prev)
    idx, zq_re, zq_im, loss_sum = _main(
        z_re, z_im, bias, codebook, cnorm, hi, mid, lo)

    mean = loss_sum[0, 0] / jnp.float32(_TOK * _D)
    loss = mean + jnp.float32(_COMMIT) * mean

    z_out = lax.complex(zq_re.reshape(_B, _T, _LATENT),
                        zq_im.reshape(_B, _T, _LATENT))
    return (z_out, loss, idx.reshape(_B, _T))


# speedup vs baseline: 1.3206x; 1.0196x over previous
import functools

import jax
import jax.numpy as jnp
from jax import lax
from jax.experimental import pallas as pl
from jax.experimental.pallas import tpu as pltpu
from jax.experimental.pallas import tpu_sc as plsc

_BIAS_SCALE = 0.8
_COMMIT = 0.01

_B, _T, _LATENT, _K = 32, 1024, 128, 1024
_D = 2 * _LATENT
_TOK = _B * _T
_TOK_TILE = 512
_N_TILES = _TOK // _TOK_TILE

_NC, _NS = 2, 16
_NW = _NC * _NS
_BPW = _TOK // _NW
_CH = 64
_NCH = _BPW // _CH


def _prep_body(adj_ref, cb_ref, s_ref, cnorm_ref, hi_ref, mid_ref):
    adj = adj_ref[...]
    s_ref[...] = _BIAS_SCALE * jax.nn.sigmoid(adj)
    cb = cb_ref[...]
    q = cb * cb
    ones = jnp.ones((1, _D), jnp.float32)
    cnorm_ref[...] = lax.dot_general(
        ones, q, (((1,), (1,)), ((), ())),
        precision=lax.Precision.HIGHEST,
        preferred_element_type=jnp.float32)
    hi = cb.astype(jnp.bfloat16)
    r1 = cb - hi.astype(jnp.float32)
    mid = r1.astype(jnp.bfloat16)
    hi_ref[...] = hi
    mid_ref[...] = mid


def _prep(adjacency, codebook):
    return pl.pallas_call(
        _prep_body,
        out_shape=(
            jax.ShapeDtypeStruct((_K, _K), jnp.float32),
            jax.ShapeDtypeStruct((1, _K), jnp.float32),
            jax.ShapeDtypeStruct((_K, _D), jnp.bfloat16),
            jax.ShapeDtypeStruct((_K, _D), jnp.bfloat16),
        ),
    )(adjacency, codebook)


@functools.partial(
    pl.kernel,
    mesh=plsc.VectorSubcoreMesh(core_axis_name="c", subcore_axis_name="s"),
    out_type=jax.ShapeDtypeStruct((_TOK, _K), jnp.float32),
    scratch_types=[
        pltpu.VMEM((_CH,), jnp.int32),
        pltpu.VMEM((_CH, _K), jnp.float32),
        pltpu.SemaphoreType.DMA,
    ],
)
def _sc_gather(s_hbm, idx_hbm, out_hbm, idx_v, rows_v, sem):
    wid = lax.axis_index("s") * _NC + lax.axis_index("c")
    base = wid * _BPW

    def body(c, carry):
        off = base + c * _CH
        pltpu.sync_copy(idx_hbm.at[pl.ds(off, _CH)], idx_v)
        pltpu.async_copy(s_hbm.at[idx_v], rows_v, sem).wait()
        pltpu.sync_copy(rows_v, out_hbm.at[pl.ds(off, _CH)])
        return carry

    lax.fori_loop(0, _NCH, body, 0)


def _main_body(zre_ref, zim_ref, bias_ref, cb_ref, cnorm_ref,
               hi_ref, mid_ref,
               idx_ref, zqre_ref, zqim_ref, loss_ref):
    zt = jnp.concatenate([zre_ref[...], zim_ref[...]], axis=1)
    m2 = lax.dot_general(
        zt, cb_ref[...], (((1,), (1,)), ((), ())),
        preferred_element_type=jnp.float32)
    zsum = jnp.sum(zt * zt, axis=1, keepdims=True)
    bt = bias_ref[...]
    d = (zsum + cnorm_ref[...]) - 2.0 * m2
    d = d - bt

    dmin = jnp.min(d, axis=1, keepdims=True)
    iota = lax.broadcasted_iota(jnp.int32, (_TOK_TILE, _K), 1)
    masked = jnp.where(d == dmin, iota, _K)
    idx = jnp.min(masked, axis=1, keepdims=True)
    idx_ref[...] = idx

    idxb = jnp.broadcast_to(idx, (_TOK_TILE, _K))
    onehot = jnp.where(iota == idxb, 1.0, 0.0)
    onehot_bf = onehot.astype(jnp.bfloat16)
    dims = (((1,), (0,)), ((), ()))
    zq = (lax.dot_general(onehot_bf, hi_ref[...], dims,
                          preferred_element_type=jnp.float32)
          + lax.dot_general(onehot_bf, mid_ref[...], dims,
                            preferred_element_type=jnp.float32))
    zqre_ref[...] = zq[:, :_LATENT]
    zqim_ref[...] = zq[:, _LATENT:]
    diff = zq - zt
    lpart = jnp.sum(diff * diff, keepdims=True)

    @pl.when(pl.program_id(0) == 0)
    def _():
        loss_ref[...] = jnp.zeros((1, 1), jnp.float32)

    loss_ref[...] += lpart


def _main(z_re, z_im, bias, codebook, cnorm, hi, mid):
    return pl.pallas_call(
        _main_body,
        grid=(_N_TILES,),
        in_specs=[
            pl.BlockSpec((_TOK_TILE, _LATENT), lambda i: (i, 0)),
            pl.BlockSpec((_TOK_TILE, _LATENT), lambda i: (i, 0)),
            pl.BlockSpec((_TOK_TILE, _K), lambda i: (i, 0)),
            pl.BlockSpec((_K, _D), lambda i: (0, 0)),
            pl.BlockSpec((1, _K), lambda i: (0, 0)),
            pl.BlockSpec((_K, _D), lambda i: (0, 0)),
            pl.BlockSpec((_K, _D), lambda i: (0, 0)),
        ],
        out_specs=[
            pl.BlockSpec((_TOK_TILE, 1), lambda i: (i, 0)),
            pl.BlockSpec((_TOK_TILE, _LATENT), lambda i: (i, 0)),
            pl.BlockSpec((_TOK_TILE, _LATENT), lambda i: (i, 0)),
            pl.BlockSpec((1, 1), lambda i: (0, 0)),
        ],
        out_shape=(
            jax.ShapeDtypeStruct((_TOK, 1), jnp.int32),
            jax.ShapeDtypeStruct((_TOK, _LATENT), jnp.float32),
            jax.ShapeDtypeStruct((_TOK, _LATENT), jnp.float32),
            jax.ShapeDtypeStruct((1, 1), jnp.float32),
        ),
    )(z_re, z_im, bias, codebook, cnorm, hi, mid)


def kernel(z, prev_symbol_idx, codebook, adjacency):
    z_re = z.real.reshape(_TOK, _LATENT)
    z_im = z.imag.reshape(_TOK, _LATENT)
    prev = prev_symbol_idx.reshape(_TOK).astype(jnp.int32)

    s, cnorm, hi, mid = _prep(adjacency, codebook)
    bias = _sc_gather(s, prev)
    idx, zq_re, zq_im, loss_sum = _main(
        z_re, z_im, bias, codebook, cnorm, hi, mid)

    mean = loss_sum[0, 0] / jnp.float32(_TOK * _D)
    loss = mean + jnp.float32(_COMMIT) * mean

    z_out = lax.complex(zq_re.reshape(_B, _T, _LATENT),
                        zq_im.reshape(_B, _T, _LATENT))
    return (z_out, loss, idx.reshape(_B, _T))


# speedup vs baseline: 1.3574x; 1.0279x over previous
import functools

import jax
import jax.numpy as jnp
from jax import lax
from jax.experimental import pallas as pl
from jax.experimental.pallas import tpu as pltpu
from jax.experimental.pallas import tpu_sc as plsc

_BIAS_SCALE = 0.8
_COMMIT = 0.01

_B, _T, _LATENT, _K = 32, 1024, 128, 1024
_D = 2 * _LATENT
_TOK = _B * _T
_TOK_TILE = 1024
_N_TILES = _TOK // _TOK_TILE

_NC, _NS = 2, 16
_NW = _NC * _NS
_BPW = _TOK // _NW
_CH = 64
_NCH = _BPW // _CH


def _prep_body(adj_ref, cb_ref, s_ref, cnorm_ref, hi_ref, mid_ref):
    adj = adj_ref[...]
    s_ref[...] = _BIAS_SCALE * jax.nn.sigmoid(adj)
    cb = cb_ref[...]
    q = cb * cb
    ones = jnp.ones((1, _D), jnp.float32)
    cnorm_ref[...] = lax.dot_general(
        ones, q, (((1,), (1,)), ((), ())),
        precision=lax.Precision.HIGHEST,
        preferred_element_type=jnp.float32)
    hi = cb.astype(jnp.bfloat16)
    r1 = cb - hi.astype(jnp.float32)
    mid = r1.astype(jnp.bfloat16)
    hi_ref[...] = hi
    mid_ref[...] = mid


def _prep(adjacency, codebook):
    return pl.pallas_call(
        _prep_body,
        out_shape=(
            jax.ShapeDtypeStruct((_K, _K), jnp.float32),
            jax.ShapeDtypeStruct((1, _K), jnp.float32),
            jax.ShapeDtypeStruct((_K, _D), jnp.bfloat16),
            jax.ShapeDtypeStruct((_K, _D), jnp.bfloat16),
        ),
    )(adjacency, codebook)


@functools.partial(
    pl.kernel,
    mesh=plsc.VectorSubcoreMesh(core_axis_name="c", subcore_axis_name="s"),
    out_type=jax.ShapeDtypeStruct((_TOK, _K), jnp.float32),
    scratch_types=[
        pltpu.VMEM((_CH,), jnp.int32),
        pltpu.VMEM((_CH, _K), jnp.float32),
        pltpu.SemaphoreType.DMA,
    ],
)
def _sc_gather(s_hbm, idx_hbm, out_hbm, idx_v, rows_v, sem):
    wid = lax.axis_index("s") * _NC + lax.axis_index("c")
    base = wid * _BPW

    def body(c, carry):
        off = base + c * _CH
        pltpu.sync_copy(idx_hbm.at[pl.ds(off, _CH)], idx_v)
        pltpu.async_copy(s_hbm.at[idx_v], rows_v, sem).wait()
        pltpu.sync_copy(rows_v, out_hbm.at[pl.ds(off, _CH)])
        return carry

    lax.fori_loop(0, _NCH, body, 0)


def _main_body(zre_ref, zim_ref, bias_ref, cb_ref, cnorm_ref,
               hi_ref, mid_ref,
               idx_ref, zqre_ref, zqim_ref, loss_ref):
    zt = jnp.concatenate([zre_ref[...], zim_ref[...]], axis=1)
    m2 = lax.dot_general(
        zt, cb_ref[...], (((1,), (1,)), ((), ())),
        preferred_element_type=jnp.float32)
    zsum = jnp.sum(zt * zt, axis=1, keepdims=True)
    bt = bias_ref[...]
    d = (zsum + cnorm_ref[...]) - 2.0 * m2
    d = d - bt

    dmin = jnp.min(d, axis=1, keepdims=True)
    iota = lax.broadcasted_iota(jnp.int32, (_TOK_TILE, _K), 1)
    masked = jnp.where(d == dmin, iota, _K)
    idx = jnp.min(masked, axis=1, keepdims=True)
    idx_ref[...] = idx

    idxb = jnp.broadcast_to(idx, (_TOK_TILE, _K))
    onehot = jnp.where(iota == idxb, 1.0, 0.0)
    onehot_bf = onehot.astype(jnp.bfloat16)
    dims = (((1,), (0,)), ((), ()))
    zq = (lax.dot_general(onehot_bf, hi_ref[...], dims,
                          preferred_element_type=jnp.float32)
          + lax.dot_general(onehot_bf, mid_ref[...], dims,
                            preferred_element_type=jnp.float32))
    zqre_ref[...] = zq[:, :_LATENT]
    zqim_ref[...] = zq[:, _LATENT:]
    diff = zq - zt
    lpart = jnp.sum(diff * diff, keepdims=True)

    @pl.when(pl.program_id(0) == 0)
    def _():
        loss_ref[...] = jnp.zeros((1, 1), jnp.float32)

    loss_ref[...] += lpart


def _main(z_re, z_im, bias, codebook, cnorm, hi, mid):
    return pl.pallas_call(
        _main_body,
        grid=(_N_TILES,),
        in_specs=[
            pl.BlockSpec((_TOK_TILE, _LATENT), lambda i: (i, 0)),
            pl.BlockSpec((_TOK_TILE, _LATENT), lambda i: (i, 0)),
            pl.BlockSpec((_TOK_TILE, _K), lambda i: (i, 0)),
            pl.BlockSpec((_K, _D), lambda i: (0, 0)),
            pl.BlockSpec((1, _K), lambda i: (0, 0)),
            pl.BlockSpec((_K, _D), lambda i: (0, 0)),
            pl.BlockSpec((_K, _D), lambda i: (0, 0)),
        ],
        out_specs=[
            pl.BlockSpec((_TOK_TILE, 1), lambda i: (i, 0)),
            pl.BlockSpec((_TOK_TILE, _LATENT), lambda i: (i, 0)),
            pl.BlockSpec((_TOK_TILE, _LATENT), lambda i: (i, 0)),
            pl.BlockSpec((1, 1), lambda i: (0, 0)),
        ],
        out_shape=(
            jax.ShapeDtypeStruct((_TOK, 1), jnp.int32),
            jax.ShapeDtypeStruct((_TOK, _LATENT), jnp.float32),
            jax.ShapeDtypeStruct((_TOK, _LATENT), jnp.float32),
            jax.ShapeDtypeStruct((1, 1), jnp.float32),
        ),
    )(z_re, z_im, bias, codebook, cnorm, hi, mid)


def kernel(z, prev_symbol_idx, codebook, adjacency):
    z_re = z.real.reshape(_TOK, _LATENT)
    z_im = z.imag.reshape(_TOK, _LATENT)
    prev = prev_symbol_idx.reshape(_TOK).astype(jnp.int32)

    s, cnorm, hi, mid = _prep(adjacency, codebook)
    bias = _sc_gather(s, prev)
    idx, zq_re, zq_im, loss_sum = _main(
        z_re, z_im, bias, codebook, cnorm, hi, mid)

    mean = loss_sum[0, 0] / jnp.float32(_TOK * _D)
    loss = mean + jnp.float32(_COMMIT) * mean

    z_out = lax.complex(zq_re.reshape(_B, _T, _LATENT),
                        zq_im.reshape(_B, _T, _LATENT))
    return (z_out, loss, idx.reshape(_B, _T))


# speedup vs baseline: 1.3745x; 1.0127x over previous
import functools

import jax
import jax.numpy as jnp
from jax import lax
from jax.experimental import pallas as pl
from jax.experimental.pallas import tpu as pltpu
from jax.experimental.pallas import tpu_sc as plsc

_BIAS_SCALE = 0.8
_COMMIT = 0.01

_B, _T, _LATENT, _K = 32, 1024, 128, 1024
_D = 2 * _LATENT
_TOK = _B * _T
_TOK_TILE = 2048
_N_TILES = _TOK // _TOK_TILE

_NC, _NS = 2, 16
_NW = _NC * _NS
_BPW = _TOK // _NW
_CH = 64
_NCH = _BPW // _CH


def _prep_body(adj_ref, cb_ref, s_ref, cnorm_ref, hi_ref, mid_ref):
    adj = adj_ref[...]
    s_ref[...] = _BIAS_SCALE * jax.nn.sigmoid(adj)
    cb = cb_ref[...]
    q = cb * cb
    ones = jnp.ones((1, _D), jnp.float32)
    cnorm_ref[...] = lax.dot_general(
        ones, q, (((1,), (1,)), ((), ())),
        precision=lax.Precision.HIGHEST,
        preferred_element_type=jnp.float32)
    hi = cb.astype(jnp.bfloat16)
    r1 = cb - hi.astype(jnp.float32)
    mid = r1.astype(jnp.bfloat16)
    hi_ref[...] = hi
    mid_ref[...] = mid


def _prep(adjacency, codebook):
    return pl.pallas_call(
        _prep_body,
        out_shape=(
            jax.ShapeDtypeStruct((_K, _K), jnp.float32),
            jax.ShapeDtypeStruct((1, _K), jnp.float32),
            jax.ShapeDtypeStruct((_K, _D), jnp.bfloat16),
            jax.ShapeDtypeStruct((_K, _D), jnp.bfloat16),
        ),
    )(adjacency, codebook)


@functools.partial(
    pl.kernel,
    mesh=plsc.VectorSubcoreMesh(core_axis_name="c", subcore_axis_name="s"),
    out_type=jax.ShapeDtypeStruct((_TOK, _K), jnp.float32),
    scratch_types=[
        pltpu.VMEM((_CH,), jnp.int32),
        pltpu.VMEM((_CH, _K), jnp.float32),
        pltpu.SemaphoreType.DMA,
    ],
)
def _sc_gather(s_hbm, idx_hbm, out_hbm, idx_v, rows_v, sem):
    wid = lax.axis_index("s") * _NC + lax.axis_index("c")
    base = wid * _BPW

    def body(c, carry):
        off = base + c * _CH
        pltpu.sync_copy(idx_hbm.at[pl.ds(off, _CH)], idx_v)
        pltpu.async_copy(s_hbm.at[idx_v], rows_v, sem).wait()
        pltpu.sync_copy(rows_v, out_hbm.at[pl.ds(off, _CH)])
        return carry

    lax.fori_loop(0, _NCH, body, 0)


def _main_body(zre_ref, zim_ref, bias_ref, cb_ref, cnorm_ref,
               hi_ref, mid_ref,
               idx_ref, zqre_ref, zqim_ref, loss_ref):
    zt = jnp.concatenate([zre_ref[...], zim_ref[...]], axis=1)
    m2 = lax.dot_general(
        zt, cb_ref[...], (((1,), (1,)), ((), ())),
        preferred_element_type=jnp.float32)
    zsum = jnp.sum(zt * zt, axis=1, keepdims=True)
    bt = bias_ref[...]
    d = (zsum + cnorm_ref[...]) - 2.0 * m2
    d = d - bt

    dmin = jnp.min(d, axis=1, keepdims=True)
    iota = lax.broadcasted_iota(jnp.int32, (_TOK_TILE, _K), 1)
    masked = jnp.where(d == dmin, iota, _K)
    idx = jnp.min(masked, axis=1, keepdims=True)
    idx_ref[...] = idx

    idxb = jnp.broadcast_to(idx, (_TOK_TILE, _K))
    onehot = jnp.where(iota == idxb, 1.0, 0.0)
    onehot_bf = onehot.astype(jnp.bfloat16)
    dims = (((1,), (0,)), ((), ()))
    zq = (lax.dot_general(onehot_bf, hi_ref[...], dims,
                          preferred_element_type=jnp.float32)
          + lax.dot_general(onehot_bf, mid_ref[...], dims,
                            preferred_element_type=jnp.float32))
    zqre_ref[...] = zq[:, :_LATENT]
    zqim_ref[...] = zq[:, _LATENT:]
    diff = zq - zt
    lpart = jnp.sum(diff * diff, keepdims=True)

    @pl.when(pl.program_id(0) == 0)
    def _():
        loss_ref[...] = jnp.zeros((1, 1), jnp.float32)

    loss_ref[...] += lpart


def _main(z_re, z_im, bias, codebook, cnorm, hi, mid):
    return pl.pallas_call(
        _main_body,
        grid=(_N_TILES,),
        in_specs=[
            pl.BlockSpec((_TOK_TILE, _LATENT), lambda i: (i, 0)),
            pl.BlockSpec((_TOK_TILE, _LATENT), lambda i: (i, 0)),
            pl.BlockSpec((_TOK_TILE, _K), lambda i: (i, 0)),
            pl.BlockSpec((_K, _D), lambda i: (0, 0)),
            pl.BlockSpec((1, _K), lambda i: (0, 0)),
            pl.BlockSpec((_K, _D), lambda i: (0, 0)),
            pl.BlockSpec((_K, _D), lambda i: (0, 0)),
        ],
        out_specs=[
            pl.BlockSpec((_TOK_TILE, 1), lambda i: (i, 0)),
            pl.BlockSpec((_TOK_TILE, _LATENT), lambda i: (i, 0)),
            pl.BlockSpec((_TOK_TILE, _LATENT), lambda i: (i, 0)),
            pl.BlockSpec((1, 1), lambda i: (0, 0)),
        ],
        out_shape=(
            jax.ShapeDtypeStruct((_TOK, 1), jnp.int32),
            jax.ShapeDtypeStruct((_TOK, _LATENT), jnp.float32),
            jax.ShapeDtypeStruct((_TOK, _LATENT), jnp.float32),
            jax.ShapeDtypeStruct((1, 1), jnp.float32),
        ),
    )(z_re, z_im, bias, codebook, cnorm, hi, mid)


def kernel(z, prev_symbol_idx, codebook, adjacency):
    z_re = z.real.reshape(_TOK, _LATENT)
    z_im = z.imag.reshape(_TOK, _LATENT)
    prev = prev_symbol_idx.reshape(_TOK).astype(jnp.int32)

    s, cnorm, hi, mid = _prep(adjacency, codebook)
    bias = _sc_gather(s, prev)
    idx, zq_re, zq_im, loss_sum = _main(
        z_re, z_im, bias, codebook, cnorm, hi, mid)

    mean = loss_sum[0, 0] / jnp.float32(_TOK * _D)
    loss = mean + jnp.float32(_COMMIT) * mean

    z_out = lax.complex(zq_re.reshape(_B, _T, _LATENT),
                        zq_im.reshape(_B, _T, _LATENT))
    return (z_out, loss, idx.reshape(_B, _T))
